# baseline (speedup 1.0000x reference)
import functools

import jax
import jax.numpy as jnp
from jax import lax
from jax.experimental import pallas as pl
from jax.experimental.pallas import tpu as pltpu
from jax.experimental.pallas import tpu_sc as plsc

N = 10000
E = 160000
D = 256
DQ = 64
NT = 16
EPT = E // NT
CH = 80
NCHUNK = EPT // CH
CHP = 128
NEPP = 10240
NP = 10240
WQT = NP // NT
WCH = 128
NWC = WQT // WCH
DEGW = 16
RB = 5
PD = 2
NCHP = NEPP // CHP
DUMPROW = 10200

_mesh = plsc.VectorSubcoreMesh(core_axis_name="c", subcore_axis_name="s")


def _zero16():
    return jnp.zeros((16,), jnp.float32)


def _ones16():
    return jnp.ones((16,), jnp.float32)


@functools.partial(
    pl.kernel,
    out_type=(
        jax.ShapeDtypeStruct((NP, DEGW), jnp.float32),
        jax.ShapeDtypeStruct((NP, DEGW), jnp.float32),
    ),
    mesh=_mesh,
    compiler_params=pltpu.CompilerParams(use_tc_tiling_on_sc=False),
    scratch_types=[
        pltpu.VMEM((NCHUNK, CH), jnp.int32),
        pltpu.VMEM((CH, DEGW), jnp.float32),
        pltpu.VMEM((WCH, DEGW), jnp.float32),
        pltpu.VMEM_SHARED((NP, DEGW), jnp.float32),
    ],
)
def _deg_kernel(graph4, dout, din, idxbuf, onesbuf, stage, slab):
    c = lax.axis_index("c")
    s = lax.axis_index("s")

    def _fill(i, _):
        stage[i, :] = _zero16()
        return 0

    lax.fori_loop(0, WCH, _fill, 0)

    def _fill1(i, _):
        onesbuf[i, :] = _ones16()
        return 0

    lax.fori_loop(0, CH, _fill1, 0)

    def _zero(w, _):
        pltpu.sync_copy(stage, slab.at[pl.ds(s * WQT + w * WCH, WCH)])
        return 0

    lax.fori_loop(0, NWC, _zero, 0)
    pltpu.sync_copy(graph4.at[c, s], idxbuf)
    plsc.subcore_barrier()

    def _scatter(k, _):
        pltpu.sync_copy(onesbuf, slab.at[idxbuf.at[k]], add=True)
        return 0

    lax.fori_loop(0, NCHUNK, _scatter, 0)
    plsc.subcore_barrier()

    def _writeout(out_ref):
        def _w(w, _):
            r0 = s * WQT + w * WCH
            pltpu.sync_copy(slab.at[pl.ds(r0, WCH)], stage)
            pltpu.sync_copy(stage, out_ref.at[pl.ds(r0, WCH)])
            return 0

        lax.fori_loop(0, NWC, _w, 0)

    @pl.when(c == 0)
    def _():
        _writeout(dout)

    @pl.when(c == 1)
    def _():
        _writeout(din)


@functools.partial(
    pl.kernel,
    out_type=tuple(
        jax.ShapeDtypeStruct((NP, DQ), jnp.float32) for _ in range(4)
    ),
    mesh=_mesh,
    compiler_params=pltpu.CompilerParams(use_tc_tiling_on_sc=False),
    scratch_types=[
        pltpu.VMEM((NCHP, CHP), jnp.int32),
        pltpu.VMEM((NCHP, CHP), jnp.int32),
        pltpu.VMEM((NCHP, CHP), jnp.float32),
        pltpu.VMEM((RB, CHP, DQ), jnp.float32),
        pltpu.VMEM((WCH, DQ), jnp.float32),
        pltpu.VMEM_SHARED((NP, DQ), jnp.float32),
        pltpu.SemaphoreType.DMA,
        pltpu.SemaphoreType.DMA,
    ],
)
def _prop_kernel(t0, t1, t2, t3, src3, dst3, ew3, o0, o1, o2, o3,
                 srcbuf, dstbuf, ewbuf, rows, stage, slab, gsem, ssem):
    c = lax.axis_index("c")
    s = lax.axis_index("s")

    def _zstage(i, _):
        def _zcol(q, _2):
            stage[i, pl.ds(q * 16, 16)] = _zero16()
            return 0

        lax.fori_loop(0, DQ // 16, _zcol, 0, unroll=True)
        return 0

    lax.fori_loop(0, WCH, _zstage, 0)
    pltpu.sync_copy(src3.at[s], srcbuf)
    pltpu.sync_copy(dst3.at[s], dstbuf)
    pltpu.sync_copy(ew3.at[s], ewbuf)

    def _quarter(tab_ref, out_ref):
        def _zero(w, _):
            pltpu.sync_copy(stage, slab.at[pl.ds(s * WQT + w * WCH, WCH)])
            return 0

        lax.fori_loop(0, NWC, _zero, 0)
        plsc.subcore_barrier()

        def _gather_start(k, b):
            pltpu.make_async_copy(
                tab_ref.at[srcbuf.at[k]], rows.at[b], gsem).start()

        def _gather_wait(k, b):
            pltpu.make_async_copy(
                tab_ref.at[srcbuf.at[k]], rows.at[b], gsem).wait()

        def _scatter_start(k, b):
            pltpu.make_async_copy(
                rows.at[b], slab.at[dstbuf.at[k]], ssem).start(add=True)

        def _scatter_wait(k, b):
            pltpu.make_async_copy(
                rows.at[b], slab.at[dstbuf.at[k]], ssem).wait()

        for b in range(PD):
            _gather_start(b, b)

        def _group(g, _):
            for b in range(RB):
                k = g * RB + b

                @pl.when(k + PD < NCHP)
                def _(k=k, b=b):
                    nb = (b + PD) % RB

                    @pl.when(k + PD - RB >= 0)
                    def _():
                        _scatter_wait(k + PD - RB, nb)

                    _gather_start(k + PD, nb)

                _gather_wait(k, b)

                @plsc.parallel_loop(0, CHP // 16, unroll=2)
                def _sgrp(jg, k=k, b=b):
                    wv = ewbuf[k, pl.ds(jg * 16, 16)]
                    for lane in range(16):
                        w = wv[lane]
                        j = jg * 16 + lane

                        def _scol(q, _3, j=j, w=w):
                            rows[b, j, pl.ds(q * 16, 16)] = (
                                rows[b, j, pl.ds(q * 16, 16)] * w)
                            return 0

                        lax.fori_loop(0, DQ // 16, _scol, 0, unroll=True)

                _scatter_start(k, b)
            return 0

        lax.fori_loop(0, NCHP // RB, _group, 0)
        for b in range(RB):
            k = NCHP - RB + b
            _scatter_wait(k, k % RB)
        plsc.subcore_barrier()

        def _w(w, _):
            r0 = s * WQT + w * WCH
            pltpu.sync_copy(slab.at[pl.ds(r0, WCH)], stage)
            pltpu.sync_copy(stage, out_ref.at[pl.ds(r0, WCH)])
            return 0

        lax.fori_loop(0, NWC, _w, 0)
        plsc.subcore_barrier()

        def _rz(i, _):
            def _rzc(q, _2):
                stage[i, pl.ds(q * 16, 16)] = _zero16()
                return 0

            lax.fori_loop(0, DQ // 16, _rzc, 0, unroll=True)
            return 0

        lax.fori_loop(0, WCH, _rz, 0)

    @pl.when(c == 0)
    def _():
        _quarter(t0, o0)

    @pl.when(c == 1)
    def _():
        _quarter(t2, o2)

    @pl.when(c == 0)
    def _():
        _quarter(t1, o1)

    @pl.when(c == 1)
    def _():
        _quarter(t3, o3)


BM = 2000


def _quarter_specs():
    return [pl.BlockSpec((BM, DQ), lambda i: (i, 0)) for _ in range(4)]


def _mm1_body(do_ref, x_ref, w_ref, t0_ref, t1_ref, t2_ref, t3_ref):
    r = lax.rsqrt(jnp.maximum(do_ref[:, 0:1], 1.0))
    o = jnp.dot(x_ref[...] * r, w_ref[...], preferred_element_type=jnp.float32)
    t0_ref[...] = o[:, :DQ]
    t1_ref[...] = o[:, DQ:2 * DQ]
    t2_ref[...] = o[:, 2 * DQ:3 * DQ]
    t3_ref[...] = o[:, 3 * DQ:]


_mm1_call = pl.pallas_call(
    _mm1_body,
    grid=(N // BM,),
    in_specs=[
        pl.BlockSpec((BM, DEGW), lambda i: (i, 0)),
        pl.BlockSpec((BM, D), lambda i: (i, 0)),
        pl.BlockSpec((D, D), lambda i: (0, 0)),
    ],
    out_specs=_quarter_specs(),
    out_shape=[jax.ShapeDtypeStruct((N, DQ), jnp.float32)] * 4,
)


def _mid_body(do_ref, di_ref, s0_ref, s1_ref, s2_ref, s3_ref, b1_ref, w2_ref,
              h1_ref, t0_ref, t1_ref, t2_ref, t3_ref):
    r_in = lax.rsqrt(jnp.maximum(di_ref[:, 0:1], 1.0))
    r_out = lax.rsqrt(jnp.maximum(do_ref[:, 0:1], 1.0))
    sfull = jnp.concatenate(
        [s0_ref[...], s1_ref[...], s2_ref[...], s3_ref[...]], axis=1)
    h = jnp.tanh(sfull * r_in + b1_ref[...])
    h1_ref[...] = h
    o = jnp.dot(h * r_out, w2_ref[...], preferred_element_type=jnp.float32)
    t0_ref[...] = o[:, :DQ]
    t1_ref[...] = o[:, DQ:2 * DQ]
    t2_ref[...] = o[:, 2 * DQ:3 * DQ]
    t3_ref[...] = o[:, 3 * DQ:]


_mid_call = pl.pallas_call(
    _mid_body,
    grid=(N // BM,),
    in_specs=[
        pl.BlockSpec((BM, DEGW), lambda i: (i, 0)),
        pl.BlockSpec((BM, DEGW), lambda i: (i, 0)),
    ] + _quarter_specs() + [
        pl.BlockSpec((1, D), lambda i: (0, 0)),
        pl.BlockSpec((D, D), lambda i: (0, 0)),
    ],
    out_specs=[pl.BlockSpec((BM, D), lambda i: (i, 0))] + _quarter_specs(),
    out_shape=[jax.ShapeDtypeStruct((N, D), jnp.float32)]
    + [jax.ShapeDtypeStruct((N, DQ), jnp.float32)] * 4,
)


def _out_body(di_ref, s0_ref, s1_ref, s2_ref, s3_ref, b2_ref, h2_ref):
    r_in = lax.rsqrt(jnp.maximum(di_ref[:, 0:1], 1.0))
    sfull = jnp.concatenate(
        [s0_ref[...], s1_ref[...], s2_ref[...], s3_ref[...]], axis=1)
    h2_ref[...] = jnp.tanh(sfull * r_in + b2_ref[...])


_out_call = pl.pallas_call(
    _out_body,
    grid=(N // BM,),
    in_specs=[pl.BlockSpec((BM, DEGW), lambda i: (i, 0))]
    + _quarter_specs()
    + [pl.BlockSpec((1, D), lambda i: (0, 0))],
    out_specs=pl.BlockSpec((BM, D), lambda i: (i, 0)),
    out_shape=jax.ShapeDtypeStruct((N, D), jnp.float32),
)


def kernel(graph, node_feats, edge_weight, W1, b1, W2, b2):
    graph = graph.astype(jnp.int32)
    graph4 = graph.reshape(2, NT, NCHUNK, CH)
    epad = NEPP - EPT
    src3 = jnp.concatenate(
        [graph[0].reshape(NT, EPT), jnp.zeros((NT, epad), jnp.int32)],
        axis=1).reshape(NT, NCHP, CHP)
    dst3 = jnp.concatenate(
        [graph[1].reshape(NT, EPT), jnp.full((NT, epad), DUMPROW, jnp.int32)],
        axis=1).reshape(NT, NCHP, CHP)
    ew3 = jnp.concatenate(
        [edge_weight.reshape(NT, EPT), jnp.zeros((NT, epad), jnp.float32)],
        axis=1).reshape(NT, NCHP, CHP)
    b1r = b1.reshape(1, D)
    b2r = b2.reshape(1, D)

    dout, din = _deg_kernel(graph4)
    t = _mm1_call(dout, node_feats, W1)
    s1 = _prop_kernel(*t, src3, dst3, ew3)
    h1, *t2 = _mid_call(dout, din, *s1, b1r, W2)
    s2 = _prop_kernel(*t2, src3, dst3, ew3)
    h2 = _out_call(din, *s2, b2r)
    return jnp.concatenate([h1, h2], axis=-1)


# speedup vs baseline: 1.6564x; 1.6564x over previous
import functools

import jax
import jax.numpy as jnp
from jax import lax
from jax.experimental import pallas as pl
from jax.experimental.pallas import tpu as pltpu
from jax.experimental.pallas import tpu_sc as plsc

N = 10000
E = 160000
D = 256
DQ = 64
NT = 16
EPT = E // NT
CH = 80
NCHUNK = EPT // CH
CHP = 80
NEPP = 10000
NP = 10240
WQT = NP // NT
WCH = 128
NWC = WQT // WCH
DEGW = 16
RB = 5
PD = 2
NCHP = NEPP // CHP
DUMPROW = 10200

_mesh = plsc.VectorSubcoreMesh(core_axis_name="c", subcore_axis_name="s")


def _zero16():
    return jnp.zeros((16,), jnp.float32)


def _ones16():
    return jnp.ones((16,), jnp.float32)


@functools.partial(
    pl.kernel,
    out_type=(
        jax.ShapeDtypeStruct((NP, DEGW), jnp.float32),
        jax.ShapeDtypeStruct((NP, DEGW), jnp.float32),
    ),
    mesh=_mesh,
    compiler_params=pltpu.CompilerParams(use_tc_tiling_on_sc=False),
    scratch_types=[
        pltpu.VMEM((NCHUNK, CH), jnp.int32),
        pltpu.VMEM((CH, DEGW), jnp.float32),
        pltpu.VMEM((WCH, DEGW), jnp.float32),
        pltpu.VMEM_SHARED((NP, DEGW), jnp.float32),
    ],
)
def _deg_kernel(graph4, dout, din, idxbuf, onesbuf, stage, slab):
    c = lax.axis_index("c")
    s = lax.axis_index("s")

    def _fill(i, _):
        stage[i, :] = _zero16()
        return 0

    lax.fori_loop(0, WCH, _fill, 0)

    def _fill1(i, _):
        onesbuf[i, :] = _ones16()
        return 0

    lax.fori_loop(0, CH, _fill1, 0)

    def _zero(w, _):
        pltpu.sync_copy(stage, slab.at[pl.ds(s * WQT + w * WCH, WCH)])
        return 0

    lax.fori_loop(0, NWC, _zero, 0)
    pltpu.sync_copy(graph4.at[c, s], idxbuf)
    plsc.subcore_barrier()

    def _scatter(k, _):
        pltpu.sync_copy(onesbuf, slab.at[idxbuf.at[k]], add=True)
        return 0

    lax.fori_loop(0, NCHUNK, _scatter, 0)
    plsc.subcore_barrier()

    def _writeout(out_ref):
        def _w(w, _):
            r0 = s * WQT + w * WCH
            pltpu.sync_copy(slab.at[pl.ds(r0, WCH)], stage)
            pltpu.sync_copy(stage, out_ref.at[pl.ds(r0, WCH)])
            return 0

        lax.fori_loop(0, NWC, _w, 0)

    @pl.when(c == 0)
    def _():
        _writeout(dout)

    @pl.when(c == 1)
    def _():
        _writeout(din)


@functools.partial(
    pl.kernel,
    out_type=tuple(
        jax.ShapeDtypeStruct((NP, DQ), jnp.float32) for _ in range(4)
    ),
    mesh=_mesh,
    compiler_params=pltpu.CompilerParams(use_tc_tiling_on_sc=False),
    scratch_types=[
        pltpu.VMEM((NCHP, CHP), jnp.int32),
        pltpu.VMEM((NCHP, CHP), jnp.int32),
        pltpu.VMEM((NCHP, CHP), jnp.float32),
        pltpu.VMEM((RB, CHP, DQ), jnp.float32),
        pltpu.VMEM((WCH, DQ), jnp.float32),
        pltpu.VMEM_SHARED((NP, DQ), jnp.float32),
        pltpu.SemaphoreType.DMA,
        pltpu.SemaphoreType.DMA,
    ],
)
def _prop_kernel(t0, t1, t2, t3, src3, dst3, ew3, o0, o1, o2, o3,
                 srcbuf, dstbuf, ewbuf, rows, stage, slab, gsem, ssem):
    c = lax.axis_index("c")
    s = lax.axis_index("s")

    def _zstage(i, _):
        def _zcol(q, _2):
            stage[i, pl.ds(q * 16, 16)] = _zero16()
            return 0

        lax.fori_loop(0, DQ // 16, _zcol, 0, unroll=True)
        return 0

    lax.fori_loop(0, WCH, _zstage, 0)
    pltpu.sync_copy(src3.at[s], srcbuf)
    pltpu.sync_copy(dst3.at[s], dstbuf)
    pltpu.sync_copy(ew3.at[s], ewbuf)

    def _quarter(tab_ref, out_ref):
        def _zero(w, _):
            pltpu.sync_copy(stage, slab.at[pl.ds(s * WQT + w * WCH, WCH)])
            return 0

        lax.fori_loop(0, NWC, _zero, 0)
        plsc.subcore_barrier()

        def _gather_start(k, b):
            pltpu.make_async_copy(
                tab_ref.at[srcbuf.at[k]], rows.at[b], gsem).start()

        def _gather_wait(k, b):
            pltpu.make_async_copy(
                tab_ref.at[srcbuf.at[k]], rows.at[b], gsem).wait()

        def _scatter_start(k, b):
            pltpu.make_async_copy(
                rows.at[b], slab.at[dstbuf.at[k]], ssem).start(add=True)

        def _scatter_wait(k, b):
            pltpu.make_async_copy(
                rows.at[b], slab.at[dstbuf.at[k]], ssem).wait()

        for b in range(PD):
            _gather_start(b, b)

        def _group(g, _):
            for b in range(RB):
                k = g * RB + b

                @pl.when(k + PD < NCHP)
                def _(k=k, b=b):
                    nb = (b + PD) % RB

                    @pl.when(k + PD - RB >= 0)
                    def _():
                        _scatter_wait(k + PD - RB, nb)

                    _gather_start(k + PD, nb)

                _gather_wait(k, b)

                @plsc.parallel_loop(0, CHP // 16, unroll=2)
                def _sgrp(jg, k=k, b=b):
                    wv = ewbuf[k, pl.ds(jg * 16, 16)]
                    for lane in range(16):
                        w = wv[lane]
                        j = jg * 16 + lane

                        def _scol(q, _3, j=j, w=w):
                            rows[b, j, pl.ds(q * 16, 16)] = (
                                rows[b, j, pl.ds(q * 16, 16)] * w)
                            return 0

                        lax.fori_loop(0, DQ // 16, _scol, 0, unroll=True)

                _scatter_start(k, b)
            return 0

        lax.fori_loop(0, NCHP // RB, _group, 0)
        for b in range(RB):
            k = NCHP - RB + b
            _scatter_wait(k, k % RB)
        plsc.subcore_barrier()

        def _w(w, _):
            r0 = s * WQT + w * WCH
            pltpu.sync_copy(slab.at[pl.ds(r0, WCH)], stage)
            pltpu.sync_copy(stage, out_ref.at[pl.ds(r0, WCH)])
            return 0

        lax.fori_loop(0, NWC, _w, 0)
        plsc.subcore_barrier()

        def _rz(i, _):
            def _rzc(q, _2):
                stage[i, pl.ds(q * 16, 16)] = _zero16()
                return 0

            lax.fori_loop(0, DQ // 16, _rzc, 0, unroll=True)
            return 0

        lax.fori_loop(0, WCH, _rz, 0)

    @pl.when(c == 0)
    def _():
        _quarter(t0, o0)

    @pl.when(c == 1)
    def _():
        _quarter(t2, o2)

    @pl.when(c == 0)
    def _():
        _quarter(t1, o1)

    @pl.when(c == 1)
    def _():
        _quarter(t3, o3)


BM = 2000


def _quarter_specs():
    return [pl.BlockSpec((BM, DQ), lambda i: (i, 0)) for _ in range(4)]


def _mm1_body(do_ref, x_ref, w_ref, t0_ref, t1_ref, t2_ref, t3_ref):
    r = lax.rsqrt(jnp.maximum(do_ref[:, 0:1], 1.0))
    o = jnp.dot(x_ref[...] * r, w_ref[...], preferred_element_type=jnp.float32)
    t0_ref[...] = o[:, :DQ]
    t1_ref[...] = o[:, DQ:2 * DQ]
    t2_ref[...] = o[:, 2 * DQ:3 * DQ]
    t3_ref[...] = o[:, 3 * DQ:]


_mm1_call = pl.pallas_call(
    _mm1_body,
    grid=(N // BM,),
    in_specs=[
        pl.BlockSpec((BM, DEGW), lambda i: (i, 0)),
        pl.BlockSpec((BM, D), lambda i: (i, 0)),
        pl.BlockSpec((D, D), lambda i: (0, 0)),
    ],
    out_specs=_quarter_specs(),
    out_shape=[jax.ShapeDtypeStruct((N, DQ), jnp.float32)] * 4,
)


def _mid_body(do_ref, di_ref, s0_ref, s1_ref, s2_ref, s3_ref, b1_ref, w2_ref,
              h1_ref, t0_ref, t1_ref, t2_ref, t3_ref):
    r_in = lax.rsqrt(jnp.maximum(di_ref[:, 0:1], 1.0))
    r_out = lax.rsqrt(jnp.maximum(do_ref[:, 0:1], 1.0))
    sfull = jnp.concatenate(
        [s0_ref[...], s1_ref[...], s2_ref[...], s3_ref[...]], axis=1)
    h = jnp.tanh(sfull * r_in + b1_ref[...])
    h1_ref[...] = h
    o = jnp.dot(h * r_out, w2_ref[...], preferred_element_type=jnp.float32)
    t0_ref[...] = o[:, :DQ]
    t1_ref[...] = o[:, DQ:2 * DQ]
    t2_ref[...] = o[:, 2 * DQ:3 * DQ]
    t3_ref[...] = o[:, 3 * DQ:]


_mid_call = pl.pallas_call(
    _mid_body,
    grid=(N // BM,),
    in_specs=[
        pl.BlockSpec((BM, DEGW), lambda i: (i, 0)),
        pl.BlockSpec((BM, DEGW), lambda i: (i, 0)),
    ] + _quarter_specs() + [
        pl.BlockSpec((1, D), lambda i: (0, 0)),
        pl.BlockSpec((D, D), lambda i: (0, 0)),
    ],
    out_specs=[pl.BlockSpec((BM, D), lambda i: (i, 0))] + _quarter_specs(),
    out_shape=[jax.ShapeDtypeStruct((N, D), jnp.float32)]
    + [jax.ShapeDtypeStruct((N, DQ), jnp.float32)] * 4,
)


def _out_body(di_ref, s0_ref, s1_ref, s2_ref, s3_ref, b2_ref, h2_ref):
    r_in = lax.rsqrt(jnp.maximum(di_ref[:, 0:1], 1.0))
    sfull = jnp.concatenate(
        [s0_ref[...], s1_ref[...], s2_ref[...], s3_ref[...]], axis=1)
    h2_ref[...] = jnp.tanh(sfull * r_in + b2_ref[...])


_out_call = pl.pallas_call(
    _out_body,
    grid=(N // BM,),
    in_specs=[pl.BlockSpec((BM, DEGW), lambda i: (i, 0))]
    + _quarter_specs()
    + [pl.BlockSpec((1, D), lambda i: (0, 0))],
    out_specs=pl.BlockSpec((BM, D), lambda i: (i, 0)),
    out_shape=jax.ShapeDtypeStruct((N, D), jnp.float32),
)


def kernel(graph, node_feats, edge_weight, W1, b1, W2, b2):
    graph = graph.astype(jnp.int32)
    graph4 = graph.reshape(2, NT, NCHUNK, CH)
    epad = NEPP - EPT
    src3 = jnp.concatenate(
        [graph[0].reshape(NT, EPT), jnp.zeros((NT, epad), jnp.int32)],
        axis=1).reshape(NT, NCHP, CHP)
    dst3 = jnp.concatenate(
        [graph[1].reshape(NT, EPT), jnp.full((NT, epad), DUMPROW, jnp.int32)],
        axis=1).reshape(NT, NCHP, CHP)
    ew3 = jnp.concatenate(
        [edge_weight.reshape(NT, EPT), jnp.zeros((NT, epad), jnp.float32)],
        axis=1).reshape(NT, NCHP, CHP)
    b1r = b1.reshape(1, D)
    b2r = b2.reshape(1, D)

    dout, din = _deg_kernel(graph4)
    t = _mm1_call(dout, node_feats, W1)
    s1 = _prop_kernel(*t, src3, dst3, ew3)
    h1, *t2 = _mid_call(dout, din, *s1, b1r, W2)
    s2 = _prop_kernel(*t2, src3, dst3, ew3)
    h2 = _out_call(din, *s2, b2r)
    return jnp.concatenate([h1, h2], axis=-1)
